# baseline (speedup 1.0000x reference)
import functools

import jax
import jax.numpy as jnp
from jax import lax
from jax.experimental import pallas as pl
from jax.experimental.pallas import tpu as pltpu
from jax.experimental.pallas import tpu_sc as plsc

N_ATOMS = 10000
N_EDGES = 320000
HIDDEN = 128
VOCAB = 28
NUM_SEGMENTS = 10000

NC = 2
NS = 16
NW = NC * NS
CHUNK = 128
N_CH = N_EDGES // CHUNK
CH_PER_W = N_CH // NW
N_TAIL = N_CH - CH_PER_W * NW
CH_ALL = CH_PER_W + 1
E_REG = CH_PER_W * CHUNK
E_PAD = CH_ALL * CHUNK
C_PAD = 10240
CYC_BLOCKS = C_PAD // 128
HIST_SP = VOCAB * C_PAD
PAD_SLOT = NUM_SEGMENTS
ZSLICE = HIST_SP // NS


def _sc_body(x_hbm, a2c_hbm, zer_hbm, hist_hbm,
             x_v, ed_v, fidx_v, ones_v, zer_v, hist_s,
             sem_in, sem_sc):
    c = lax.axis_index("c")
    s = lax.axis_index("s")
    w = c * NS + s
    tail_ch = CH_PER_W * NW + (w & 3)

    cp_x = pltpu.async_copy(x_hbm, x_v, sem_in)
    cp_r = pltpu.async_copy(
        a2c_hbm.at[:, pl.ds(pl.multiple_of(w * E_REG, CHUNK), E_REG)],
        ed_v.at[:, pl.ds(0, E_REG)], sem_in)
    cp_t = pltpu.async_copy(
        a2c_hbm.at[:, pl.ds(pl.multiple_of(tail_ch * CHUNK, CHUNK), CHUNK)],
        ed_v.at[:, pl.ds(E_REG, CHUNK)], sem_in)

    pltpu.sync_copy(zer_hbm, zer_v)
    pltpu.sync_copy(zer_v, hist_s.at[pl.ds(s * ZSLICE, ZSLICE)])

    one16 = jnp.ones((16,), jnp.float32)
    for k in range(CHUNK // 16):
        ones_v[pl.ds(k * 16, 16)] = one16

    cp_x.wait()
    cp_r.wait()
    cp_t.wait()

    plsc.subcore_barrier()

    def cbody(j, carry):
        mask = jnp.logical_and(j == CH_PER_W, w >= N_TAIL)
        for k in range(CHUNK // 16):
            s16 = ed_v[0, pl.ds(j * CHUNK + k * 16, 16)]
            d16 = ed_v[1, pl.ds(j * CHUNK + k * 16, 16)]
            t16 = plsc.load_gather(x_v, [s16])
            f16 = jnp.where(mask, jnp.int32(PAD_SLOT), t16 * C_PAD + d16)
            fidx_v[j, pl.ds(k * 16, 16)] = f16
        pltpu.async_copy(ones_v, hist_s.at[fidx_v.at[j]], sem_sc, add=True)
        return carry

    lax.fori_loop(0, CH_ALL, cbody, 0)

    def dbody(j, carry):
        pltpu.make_async_copy(ones_v, hist_s.at[fidx_v.at[0]], sem_sc).wait()
        return carry

    lax.fori_loop(0, CH_ALL, dbody, 0)

    plsc.subcore_barrier()

    pltpu.sync_copy(hist_s.at[pl.ds(s * ZSLICE, ZSLICE)],
                    zer_v.at[pl.ds(0, ZSLICE)])
    pltpu.sync_copy(zer_v.at[pl.ds(0, ZSLICE)],
                    hist_hbm.at[pl.ds(c * HIST_SP + s * ZSLICE, ZSLICE)])


_sc_hist = pl.kernel(
    _sc_body,
    out_type=jax.ShapeDtypeStruct((NC * HIST_SP,), jnp.float32),
    mesh=plsc.VectorSubcoreMesh(core_axis_name="c", subcore_axis_name="s"),
    compiler_params=pltpu.CompilerParams(needs_layout_passes=False),
    scratch_types=[
        pltpu.VMEM((N_ATOMS,), jnp.int32),
        pltpu.VMEM((2, E_PAD), jnp.int32),
        pltpu.VMEM((CH_ALL, CHUNK), jnp.int32),
        pltpu.VMEM((CHUNK,), jnp.float32),
        pltpu.VMEM((ZSLICE,), jnp.float32),
        pltpu.VMEM_SHARED((HIST_SP,), jnp.float32),
        pltpu.SemaphoreType.DMA,
        pltpu.SemaphoreType.DMA,
    ],
)


MM_GRID = 5
MM_BJ = CYC_BLOCKS // MM_GRID


def _mm_body(hist_ref, emb_ref, out_ref):
    e = emb_ref[...]
    h = hist_ref[0] + hist_ref[1]
    hw = jnp.concatenate([h[:, bj, :] for bj in range(MM_BJ)], axis=1)
    out_ref[...] = lax.dot_general(
        hw, e, (((0,), (0,)), ((), ())),
        preferred_element_type=jnp.float32)


def _tc_expand(hist4, embp):
    return pl.pallas_call(
        _mm_body,
        grid=(MM_GRID,),
        in_specs=[
            pl.BlockSpec((NC, VOCAB, MM_BJ, 128), lambda i: (0, 0, i, 0)),
            pl.BlockSpec((VOCAB, HIDDEN), lambda i: (0, 0)),
        ],
        out_specs=pl.BlockSpec((MM_BJ * 128, HIDDEN), lambda i: (i, 0)),
        out_shape=jax.ShapeDtypeStruct((NUM_SEGMENTS, HIDDEN), jnp.float32),
    )(hist4, embp)


@jax.jit
def kernel(x, atom_to_cycle, emb_weight):
    zer = jnp.zeros((ZSLICE,), jnp.float32)
    hist = _sc_hist(x, atom_to_cycle, zer)
    hist4 = hist.reshape(NC, VOCAB, CYC_BLOCKS, 128)
    return _tc_expand(hist4, emb_weight)


# speedup vs baseline: 1.0637x; 1.0637x over previous
import functools

import jax
import jax.numpy as jnp
from jax import lax
from jax.experimental import pallas as pl
from jax.experimental.pallas import tpu as pltpu
from jax.experimental.pallas import tpu_sc as plsc

N_ATOMS = 10000
N_EDGES = 320000
HIDDEN = 128
VOCAB = 28
NUM_SEGMENTS = 10000

NC = 2
NS = 16
NW = NC * NS
CHUNK = 128
N_CH = N_EDGES // CHUNK
CH_PER_W = N_CH // NW
N_TAIL = N_CH - CH_PER_W * NW
CH_ALL = CH_PER_W + 1
E_REG = CH_PER_W * CHUNK
E_PAD = CH_ALL * CHUNK
C_PAD = 10240
CYC_BLOCKS = C_PAD // 128
HIST_SP = VOCAB * C_PAD
PAD_SLOT = NUM_SEGMENTS
ZSLICE = HIST_SP // NS


def _sc_body(x_hbm, a2c_hbm, hist_hbm,
             x_v, ed_v, fidx_v, ones_v, zer_v, hist_s,
             sem_in, sem_sc):
    c = lax.axis_index("c")
    s = lax.axis_index("s")
    w = c * NS + s
    tail_ch = CH_PER_W * NW + (w & 3)

    cp_x = pltpu.async_copy(x_hbm, x_v, sem_in)
    cp_r = pltpu.async_copy(
        a2c_hbm.at[:, pl.ds(pl.multiple_of(w * E_REG, CHUNK), E_REG)],
        ed_v.at[:, pl.ds(0, E_REG)], sem_in)
    cp_t = pltpu.async_copy(
        a2c_hbm.at[:, pl.ds(pl.multiple_of(tail_ch * CHUNK, CHUNK), CHUNK)],
        ed_v.at[:, pl.ds(E_REG, CHUNK)], sem_in)

    zero16 = jnp.zeros((16,), jnp.float32)

    def zbody(i, carry):
        for k in range(8):
            zer_v[pl.ds(i * 128 + k * 16, 16)] = zero16
        return carry

    lax.fori_loop(0, ZSLICE // 128, zbody, 0)
    pltpu.sync_copy(zer_v, hist_s.at[pl.ds(s * ZSLICE, ZSLICE)])

    one16 = jnp.ones((16,), jnp.float32)
    for k in range(CHUNK // 16):
        ones_v[pl.ds(k * 16, 16)] = one16

    cp_x.wait()
    cp_r.wait()
    cp_t.wait()

    plsc.subcore_barrier()

    def cbody(j, carry):
        mask = jnp.logical_and(j == CH_PER_W, w >= N_TAIL)
        for k in range(CHUNK // 16):
            s16 = ed_v[0, pl.ds(j * CHUNK + k * 16, 16)]
            d16 = ed_v[1, pl.ds(j * CHUNK + k * 16, 16)]
            t16 = plsc.load_gather(x_v, [s16])
            f16 = jnp.where(mask, jnp.int32(PAD_SLOT), t16 * C_PAD + d16)
            fidx_v[j, pl.ds(k * 16, 16)] = f16
        pltpu.async_copy(ones_v, hist_s.at[fidx_v.at[j]], sem_sc, add=True)
        return carry

    lax.fori_loop(0, CH_ALL, cbody, 0)

    def dbody(j, carry):
        pltpu.make_async_copy(ones_v, hist_s.at[fidx_v.at[0]], sem_sc).wait()
        return carry

    lax.fori_loop(0, CH_ALL, dbody, 0)

    plsc.subcore_barrier()

    pltpu.sync_copy(hist_s.at[pl.ds(s * ZSLICE, ZSLICE)],
                    zer_v.at[pl.ds(0, ZSLICE)])
    pltpu.sync_copy(zer_v.at[pl.ds(0, ZSLICE)],
                    hist_hbm.at[pl.ds(c * HIST_SP + s * ZSLICE, ZSLICE)])


_sc_hist = pl.kernel(
    _sc_body,
    out_type=jax.ShapeDtypeStruct((NC * HIST_SP,), jnp.float32),
    mesh=plsc.VectorSubcoreMesh(core_axis_name="c", subcore_axis_name="s"),
    compiler_params=pltpu.CompilerParams(needs_layout_passes=False),
    scratch_types=[
        pltpu.VMEM((N_ATOMS,), jnp.int32),
        pltpu.VMEM((2, E_PAD), jnp.int32),
        pltpu.VMEM((CH_ALL, CHUNK), jnp.int32),
        pltpu.VMEM((CHUNK,), jnp.float32),
        pltpu.VMEM((ZSLICE,), jnp.float32),
        pltpu.VMEM_SHARED((HIST_SP,), jnp.float32),
        pltpu.SemaphoreType.DMA,
        pltpu.SemaphoreType.DMA,
    ],
)


MM_GRID = 5
MM_BJ = CYC_BLOCKS // MM_GRID


def _mm_body(hist_ref, emb_ref, out_ref):
    e = emb_ref[...]
    h = hist_ref[0] + hist_ref[1]
    hw = jnp.concatenate([h[:, bj, :] for bj in range(MM_BJ)], axis=1)
    out_ref[...] = lax.dot_general(
        hw, e, (((0,), (0,)), ((), ())),
        preferred_element_type=jnp.float32)


def _tc_expand(hist4, embp):
    return pl.pallas_call(
        _mm_body,
        grid=(MM_GRID,),
        in_specs=[
            pl.BlockSpec((NC, VOCAB, MM_BJ, 128), lambda i: (0, 0, i, 0)),
            pl.BlockSpec((VOCAB, HIDDEN), lambda i: (0, 0)),
        ],
        out_specs=pl.BlockSpec((MM_BJ * 128, HIDDEN), lambda i: (i, 0)),
        out_shape=jax.ShapeDtypeStruct((NUM_SEGMENTS, HIDDEN), jnp.float32),
    )(hist4, embp)


@jax.jit
def kernel(x, atom_to_cycle, emb_weight):
    hist = _sc_hist(x, atom_to_cycle)
    hist4 = hist.reshape(NC, VOCAB, CYC_BLOCKS, 128)
    return _tc_expand(hist4, emb_weight)
